# Initial kernel scaffold; baseline (speedup 1.0000x reference)
import jax
import jax.numpy as jnp
from jax.experimental import pallas as pl


def kernel(child_feats, child_exists, edge_type_onehot, edge_feats, edge_indices, Wc, bc, Wne, bne, Wp, bp):
    raise NotImplementedError("write your pallas kernel here")


# speedup vs baseline: 3.8443x; 3.8443x over previous
import jax
import jax.numpy as jnp
from jax import lax
from jax.experimental import pallas as pl
from jax.experimental.pallas import tpu as pltpu
from jax.experimental.pallas import tpu_sc as plsc

N = 10000
H = 128
E = 320000
CH = 80
NTILES = 32
CPT = E // (NTILES * CH)
ZC = 400
NZC = N // ZC
BE = 3200


def _node_body(child_ref, exists_ref, wc_ref, bc_ref, ws_ref, wd_ref,
               a_ref, b_ref, s0_ref, es_ref):
    cf = jnp.dot(child_ref[...], wc_ref[...], preferred_element_type=jnp.float32)
    cf = jnp.maximum(cf + bc_ref[...], 0.0) * exists_ref[...]
    a_ref[...] = jnp.dot(cf, ws_ref[...], preferred_element_type=jnp.float32)
    b_ref[...] = jnp.dot(cf, wd_ref[...], preferred_element_type=jnp.float32)
    s0_ref[...] = jnp.sum(cf, axis=0, keepdims=True)
    es_ref[...] = jnp.sum(exists_ref[...], axis=0, keepdims=True)


def _c_body(oh_ref, ef_ref, w1_ref, w2_ref, bne_ref, c_ref):
    c_ref[...] = (jnp.dot(oh_ref[...], w1_ref[...], preferred_element_type=jnp.float32)
                  + jnp.dot(ef_ref[...], w2_ref[...], preferred_element_type=jnp.float32)
                  + bne_ref[...])


def _mid_body(outp_ref, ws_ref, wd_ref, a_ref, b_ref, s_ref):
    cf = outp_ref[0] + outp_ref[1]
    a_ref[...] = jnp.dot(cf, ws_ref[...], preferred_element_type=jnp.float32)
    b_ref[...] = jnp.dot(cf, wd_ref[...], preferred_element_type=jnp.float32)
    s_ref[...] = jnp.sum(cf, axis=0, keepdims=True)


def _fin_body(outp_ref, s0_ref, s1_ref, es_ref, wp_ref, bp_ref, o_ref):
    s2 = jnp.sum(outp_ref[0] + outp_ref[1], axis=0, keepdims=True)
    p = jnp.concatenate([s0_ref[...], s1_ref[...], s2], axis=1) / es_ref[0, 0]
    o_ref[...] = jnp.maximum(
        jnp.dot(p, wp_ref[...], preferred_element_type=jnp.float32) + bp_ref[...], 0.0)


def _edge_body(a_h, b_h, c_h, s_h, d_h, out_h,
               srcc, dstc, ar, br, cr, acc, sa, sb, sc2, si):
    cid = lax.axis_index("c")
    sid = lax.axis_index("s")
    w = sid * 2 + cid
    zero16 = jnp.zeros((16,), jnp.float32)

    def zrow(r, carry):
        for j in range(8):
            ar[r, pl.ds(j * 16, 16)] = zero16
        return carry
    lax.fori_loop(0, ZC // 5, zrow, 0)

    for jj in range(2):
        k = sid + 16 * jj
        @pl.when(k < NZC)
        def _zero_chunk():
            for j in range(5):
                pltpu.sync_copy(ar, acc.at[pl.ds(k * ZC + j * (ZC // 5), ZC // 5)])
    plsc.subcore_barrier()

    def chunk(k, carry):
        base = (w * CPT + k) * CH
        ds_ = pltpu.async_copy(s_h.at[pl.ds(base, CH)], srcc, si)
        dd = pltpu.async_copy(d_h.at[pl.ds(base, CH)], dstc, si)
        dc = pltpu.async_copy(c_h.at[pl.ds(base, CH)], cr, sc2)
        ds_.wait()
        dd.wait()
        da = pltpu.async_copy(a_h.at[srcc], ar, sa)
        db = pltpu.async_copy(b_h.at[dstc], br, sb)
        da.wait()
        db.wait()
        dc.wait()

        def row(r, rc):
            for j in range(8):
                s = pl.ds(j * 16, 16)
                cr[r, s] = jnp.maximum(ar[r, s] + br[r, s] + cr[r, s], 0.0)
            return rc
        lax.fori_loop(0, CH, row, 0)
        pltpu.sync_copy(cr, acc.at[srcc], add=True)
        return carry
    lax.fori_loop(0, CPT, chunk, 0)
    plsc.subcore_barrier()
    for jj in range(2):
        k = sid + 16 * jj
        @pl.when(k < NZC)
        def _writeout():
            sl = pl.ds(k * ZC, ZC)
            pltpu.sync_copy(acc.at[sl], out_h.at[cid].at[sl])


def _edge_call(a, b, c, src2, dst2):
    mesh = plsc.VectorSubcoreMesh(core_axis_name="c", subcore_axis_name="s")
    f = pl.kernel(
        _edge_body,
        out_type=jax.ShapeDtypeStruct((2, N, H), jnp.float32),
        mesh=mesh,
        scratch_types=[
            pltpu.VMEM((CH,), jnp.int32),
            pltpu.VMEM((CH,), jnp.int32),
            pltpu.VMEM((CH, H), jnp.float32),
            pltpu.VMEM((CH, H), jnp.float32),
            pltpu.VMEM((CH, H), jnp.float32),
            pltpu.VMEM_SHARED((N, H), jnp.float32),
            pltpu.SemaphoreType.DMA,
            pltpu.SemaphoreType.DMA,
            pltpu.SemaphoreType.DMA,
            pltpu.SemaphoreType.DMA,
        ],
    )
    return f(a, b, c, src2, dst2)


def _node_call(child, exists, wc, bc2, ws, wd):
    return pl.pallas_call(
        _node_body,
        out_shape=[
            jax.ShapeDtypeStruct((N, H), jnp.float32),
            jax.ShapeDtypeStruct((N, H), jnp.float32),
            jax.ShapeDtypeStruct((1, H), jnp.float32),
            jax.ShapeDtypeStruct((1, 1), jnp.float32),
        ],
    )(child, exists, wc, bc2, ws, wd)


def _c_call(oh, ef, w1, w2, bne2):
    grid = (E // BE,)
    return pl.pallas_call(
        _c_body,
        grid=grid,
        in_specs=[
            pl.BlockSpec((BE, 4), lambda i: (i, 0)),
            pl.BlockSpec((BE, 16), lambda i: (i, 0)),
            pl.BlockSpec((4, H), lambda i: (0, 0)),
            pl.BlockSpec((16, H), lambda i: (0, 0)),
            pl.BlockSpec((1, H), lambda i: (0, 0)),
        ],
        out_specs=pl.BlockSpec((BE, H), lambda i: (i, 0)),
        out_shape=jax.ShapeDtypeStruct((E, H), jnp.float32),
    )(oh, ef, w1, w2, bne2)


def _mid_call(outp, ws, wd):
    return pl.pallas_call(
        _mid_body,
        out_shape=[
            jax.ShapeDtypeStruct((N, H), jnp.float32),
            jax.ShapeDtypeStruct((N, H), jnp.float32),
            jax.ShapeDtypeStruct((1, H), jnp.float32),
        ],
    )(outp, ws, wd)


def _fin_call(outp, s0, s1, es, wp, bp2):
    return pl.pallas_call(
        _fin_body,
        out_shape=jax.ShapeDtypeStruct((1, H), jnp.float32),
    )(outp, s0, s1, es, wp, bp2)


def kernel(child_feats, child_exists, edge_type_onehot, edge_feats, edge_indices,
           Wc, bc, Wne, bne, Wp, bp):
    child = child_feats[0]
    exists = child_exists[0]
    oh = edge_type_onehot[0]
    ef = edge_feats[0]
    src2 = edge_indices[0, :, 0].astype(jnp.int32)
    dst2 = edge_indices[0, :, 1].astype(jnp.int32)
    ws = Wne[:H]
    wd = Wne[H:2 * H]
    w1 = Wne[2 * H:2 * H + 4]
    w2 = Wne[2 * H + 4:]
    bc2 = bc.reshape(1, H)
    bne2 = bne.reshape(1, H)
    bp2 = bp.reshape(1, H)

    a1, b1, s0, es = _node_call(child, exists, Wc, bc2, ws, wd)
    c = _c_call(oh, ef, w1, w2, bne2)
    outp1 = _edge_call(a1, b1, c, src2, dst2)
    a2, b2, s1 = _mid_call(outp1, ws, wd)
    outp2 = _edge_call(a2, b2, c, src2, dst2)
    return _fin_call(outp2, s0, s1, es, Wp, bp2)


# speedup vs baseline: 4.5893x; 1.1938x over previous
import jax
import jax.numpy as jnp
from jax import lax
from jax.experimental import pallas as pl
from jax.experimental.pallas import tpu as pltpu
from jax.experimental.pallas import tpu_sc as plsc

N = 10000
H = 128
E = 320000
CH = 40
NTILES = 32
CPT = E // (NTILES * CH)
ZC = 400
NZC = N // ZC
BE = 3200


def _node_body(child_ref, exists_ref, wc_ref, bc_ref, ws_ref, wd_ref,
               a_ref, b_ref, s0_ref, es_ref):
    cf = jnp.dot(child_ref[...], wc_ref[...], preferred_element_type=jnp.float32)
    cf = jnp.maximum(cf + bc_ref[...], 0.0) * exists_ref[...]
    a_ref[...] = jnp.dot(cf, ws_ref[...], preferred_element_type=jnp.float32)
    b_ref[...] = jnp.dot(cf, wd_ref[...], preferred_element_type=jnp.float32)
    s0_ref[...] = jnp.sum(cf, axis=0, keepdims=True)
    es_ref[...] = jnp.sum(exists_ref[...], axis=0, keepdims=True)


def _c_body(oh_ref, ef_ref, w1_ref, w2_ref, bne_ref, c_ref):
    c_ref[...] = (jnp.dot(oh_ref[...], w1_ref[...], preferred_element_type=jnp.float32)
                  + jnp.dot(ef_ref[...], w2_ref[...], preferred_element_type=jnp.float32)
                  + bne_ref[...])


def _mid_body(outp_ref, ws_ref, wd_ref, a_ref, b_ref, s_ref):
    cf = outp_ref[0] + outp_ref[1]
    a_ref[...] = jnp.dot(cf, ws_ref[...], preferred_element_type=jnp.float32)
    b_ref[...] = jnp.dot(cf, wd_ref[...], preferred_element_type=jnp.float32)
    s_ref[...] = jnp.sum(cf, axis=0, keepdims=True)


def _fin_body(outp_ref, s0_ref, s1_ref, es_ref, wp_ref, bp_ref, o_ref):
    s2 = jnp.sum(outp_ref[0] + outp_ref[1], axis=0, keepdims=True)
    p = jnp.concatenate([s0_ref[...], s1_ref[...], s2], axis=1) / es_ref[0, 0]
    o_ref[...] = jnp.maximum(
        jnp.dot(p, wp_ref[...], preferred_element_type=jnp.float32) + bp_ref[...], 0.0)


def _edge_body(a_h, b_h, c_h, s_h, d_h, out_h,
               srcc0, dstc0, ar0, br0, cr0,
               srcc1, dstc1, ar1, br1, cr1,
               acc, si0, si1, sg0, sg1):
    cid = lax.axis_index("c")
    sid = lax.axis_index("s")
    w = sid * 2 + cid
    tbase = w * CPT * CH
    slots = ((srcc0, dstc0, ar0, br0, cr0, si0, sg0),
             (srcc1, dstc1, ar1, br1, cr1, si1, sg1))
    zero16 = jnp.zeros((16,), jnp.float32)

    def zrow(r, carry):
        for j in range(8):
            ar0[r, pl.ds(j * 16, 16)] = zero16
        return carry
    lax.fori_loop(0, CH, zrow, 0)

    for jj in range(2):
        k = sid + 16 * jj
        @pl.when(k < NZC)
        def _zero_chunk():
            for j in range(ZC // CH):
                pltpu.sync_copy(ar0, acc.at[pl.ds(k * ZC + j * CH, CH)])
    plsc.subcore_barrier()

    def issue_idx(kc, slot):
        srcc, dstc, _, _, _, si, _ = slot
        base = tbase + kc * CH
        pltpu.async_copy(s_h.at[pl.ds(base, CH)], srcc, si)
        pltpu.async_copy(d_h.at[pl.ds(base, CH)], dstc, si)

    def wait_idx(slot):
        srcc, dstc, _, _, _, si, _ = slot
        pltpu.make_async_copy(s_h.at[pl.ds(0, CH)], srcc, si).wait()
        pltpu.make_async_copy(d_h.at[pl.ds(0, CH)], dstc, si).wait()

    def issue_gather(kc, slot):
        srcc, dstc, ar, br, cr, _, sg = slot
        pltpu.async_copy(a_h.at[srcc], ar, sg)
        pltpu.async_copy(b_h.at[dstc], br, sg)
        pltpu.async_copy(c_h.at[pl.ds(tbase + kc * CH, CH)], cr, sg)

    def wait_gather(slot):
        srcc, dstc, ar, br, cr, _, sg = slot
        pltpu.make_async_copy(a_h.at[srcc], ar, sg).wait()
        pltpu.make_async_copy(b_h.at[dstc], br, sg).wait()
        pltpu.make_async_copy(c_h.at[pl.ds(0, CH)], cr, sg).wait()

    issue_idx(0, slots[0])
    issue_idx(1, slots[1])
    wait_idx(slots[0])
    issue_gather(0, slots[0])

    def dstep(k2, carry):
        for b in range(2):
            slot = slots[b]
            other = slots[1 - b]
            srcc, dstc, ar, br, cr, si, sg = slot
            k = 2 * k2 + b
            wait_gather(slot)

            @pl.when(k + 1 < CPT)
            def _prefetch_next():
                wait_idx(other)
                issue_gather(k + 1, other)

            def row(r, rc):
                for j in range(8):
                    s = pl.ds(j * 16, 16)
                    cr[r, s] = jnp.maximum(ar[r, s] + br[r, s] + cr[r, s], 0.0)
                return rc
            lax.fori_loop(0, CH, row, 0)
            pltpu.sync_copy(cr, acc.at[srcc], add=True)

            @pl.when(k + 2 < CPT)
            def _prefetch_idx():
                issue_idx(k + 2, slot)
        return carry
    lax.fori_loop(0, CPT // 2, dstep, 0)
    plsc.subcore_barrier()
    for jj in range(2):
        k = sid + 16 * jj
        @pl.when(k < NZC)
        def _writeout():
            sl = pl.ds(k * ZC, ZC)
            pltpu.sync_copy(acc.at[sl], out_h.at[cid].at[sl])


def _edge_call(a, b, c, src2, dst2):
    mesh = plsc.VectorSubcoreMesh(core_axis_name="c", subcore_axis_name="s")
    f = pl.kernel(
        _edge_body,
        out_type=jax.ShapeDtypeStruct((2, N, H), jnp.float32),
        mesh=mesh,
        scratch_types=(
            [pltpu.VMEM((CH,), jnp.int32),
             pltpu.VMEM((CH,), jnp.int32),
             pltpu.VMEM((CH, H), jnp.float32),
             pltpu.VMEM((CH, H), jnp.float32),
             pltpu.VMEM((CH, H), jnp.float32)] * 2
            + [pltpu.VMEM_SHARED((N, H), jnp.float32),
               pltpu.SemaphoreType.DMA,
               pltpu.SemaphoreType.DMA,
               pltpu.SemaphoreType.DMA,
               pltpu.SemaphoreType.DMA]
        ),
    )
    return f(a, b, c, src2, dst2)


def _node_call(child, exists, wc, bc2, ws, wd):
    return pl.pallas_call(
        _node_body,
        out_shape=[
            jax.ShapeDtypeStruct((N, H), jnp.float32),
            jax.ShapeDtypeStruct((N, H), jnp.float32),
            jax.ShapeDtypeStruct((1, H), jnp.float32),
            jax.ShapeDtypeStruct((1, 1), jnp.float32),
        ],
    )(child, exists, wc, bc2, ws, wd)


def _c_call(oh, ef, w1, w2, bne2):
    grid = (E // BE,)
    return pl.pallas_call(
        _c_body,
        grid=grid,
        in_specs=[
            pl.BlockSpec((BE, 4), lambda i: (i, 0)),
            pl.BlockSpec((BE, 16), lambda i: (i, 0)),
            pl.BlockSpec((4, H), lambda i: (0, 0)),
            pl.BlockSpec((16, H), lambda i: (0, 0)),
            pl.BlockSpec((1, H), lambda i: (0, 0)),
        ],
        out_specs=pl.BlockSpec((BE, H), lambda i: (i, 0)),
        out_shape=jax.ShapeDtypeStruct((E, H), jnp.float32),
    )(oh, ef, w1, w2, bne2)


def _mid_call(outp, ws, wd):
    return pl.pallas_call(
        _mid_body,
        out_shape=[
            jax.ShapeDtypeStruct((N, H), jnp.float32),
            jax.ShapeDtypeStruct((N, H), jnp.float32),
            jax.ShapeDtypeStruct((1, H), jnp.float32),
        ],
    )(outp, ws, wd)


def _fin_call(outp, s0, s1, es, wp, bp2):
    return pl.pallas_call(
        _fin_body,
        out_shape=jax.ShapeDtypeStruct((1, H), jnp.float32),
    )(outp, s0, s1, es, wp, bp2)


def kernel(child_feats, child_exists, edge_type_onehot, edge_feats, edge_indices,
           Wc, bc, Wne, bne, Wp, bp):
    child = child_feats[0]
    exists = child_exists[0]
    oh = edge_type_onehot[0]
    ef = edge_feats[0]
    src2 = edge_indices[0, :, 0].astype(jnp.int32)
    dst2 = edge_indices[0, :, 1].astype(jnp.int32)
    ws = Wne[:H]
    wd = Wne[H:2 * H]
    w1 = Wne[2 * H:2 * H + 4]
    w2 = Wne[2 * H + 4:]
    bc2 = bc.reshape(1, H)
    bne2 = bne.reshape(1, H)
    bp2 = bp.reshape(1, H)

    a1, b1, s0, es = _node_call(child, exists, Wc, bc2, ws, wd)
    c = _c_call(oh, ef, w1, w2, bne2)
    outp1 = _edge_call(a1, b1, c, src2, dst2)
    a2, b2, s1 = _mid_call(outp1, ws, wd)
    outp2 = _edge_call(a2, b2, c, src2, dst2)
    return _fin_call(outp2, s0, s1, es, Wp, bp2)


# speedup vs baseline: 5.9278x; 1.2917x over previous
import jax
import jax.numpy as jnp
from jax import lax
from jax.experimental import pallas as pl
from jax.experimental.pallas import tpu as pltpu
from jax.experimental.pallas import tpu_sc as plsc

N = 10000
H = 128
E = 320000
CH = 40
NTILES = 32
CPT = E // (NTILES * CH)
NR = 3
NI = 6
ZC = 400
NZC = N // ZC
BE = 3200


def _node_body(child_ref, exists_ref, wc_ref, bc_ref, ws_ref, wd_ref,
               a_ref, b_ref, s0_ref, es_ref):
    cf = jnp.dot(child_ref[...], wc_ref[...], preferred_element_type=jnp.float32)
    cf = jnp.maximum(cf + bc_ref[...], 0.0) * exists_ref[...]
    a_ref[...] = jnp.dot(cf, ws_ref[...], preferred_element_type=jnp.float32)
    b_ref[...] = jnp.dot(cf, wd_ref[...], preferred_element_type=jnp.float32)
    s0_ref[...] = jnp.sum(cf, axis=0, keepdims=True)
    es_ref[...] = jnp.sum(exists_ref[...], axis=0, keepdims=True)


def _c_body(oh_ref, ef_ref, w1_ref, w2_ref, bne_ref, c_ref):
    c_ref[...] = (jnp.dot(oh_ref[...], w1_ref[...], preferred_element_type=jnp.float32)
                  + jnp.dot(ef_ref[...], w2_ref[...], preferred_element_type=jnp.float32)
                  + bne_ref[...])


def _mid_body(outp_ref, ws_ref, wd_ref, a_ref, b_ref, s_ref):
    cf = outp_ref[0] + outp_ref[1]
    a_ref[...] = jnp.dot(cf, ws_ref[...], preferred_element_type=jnp.float32)
    b_ref[...] = jnp.dot(cf, wd_ref[...], preferred_element_type=jnp.float32)
    s_ref[...] = jnp.sum(cf, axis=0, keepdims=True)


def _fin_body(outp_ref, s0_ref, s1_ref, es_ref, wp_ref, bp_ref, o_ref):
    s2 = jnp.sum(outp_ref[0] + outp_ref[1], axis=0, keepdims=True)
    p = jnp.concatenate([s0_ref[...], s1_ref[...], s2], axis=1) / es_ref[0, 0]
    o_ref[...] = jnp.maximum(
        jnp.dot(p, wp_ref[...], preferred_element_type=jnp.float32) + bp_ref[...], 0.0)


def _edge_body(a_h, b_h, c_h, s_h, d_h, out_h, *refs):
    idx_slots = []
    for i in range(NI):
        idx_slots.append(tuple(refs[3 * i:3 * i + 3]))
    row_slots = []
    for i in range(NR):
        row_slots.append(tuple(refs[3 * NI + 4 * i:3 * NI + 4 * i + 4]))
    acc = refs[3 * NI + 4 * NR]

    cid = lax.axis_index("c")
    sid = lax.axis_index("s")
    w = sid * 2 + cid
    tbase = w * CPT * CH
    zero16 = jnp.zeros((16,), jnp.float32)
    zbuf = row_slots[0][0]

    def zrow(r, carry):
        for j in range(8):
            zbuf[r, pl.ds(j * 16, 16)] = zero16
        return carry
    lax.fori_loop(0, CH, zrow, 0)

    for jj in range(2):
        k = sid + 16 * jj
        @pl.when(k < NZC)
        def _zero_chunk():
            for j in range(ZC // CH):
                pltpu.sync_copy(zbuf, acc.at[pl.ds(k * ZC + j * CH, CH)])
    plsc.subcore_barrier()

    def issue_idx(kc, islot):
        srcc, dstc, si = islot
        base = tbase + kc * CH
        pltpu.async_copy(s_h.at[pl.ds(base, CH)], srcc, si)
        pltpu.async_copy(d_h.at[pl.ds(base, CH)], dstc, si)

    def wait_idx(islot):
        srcc, dstc, si = islot
        pltpu.make_async_copy(s_h.at[pl.ds(0, CH)], srcc, si).wait()
        pltpu.make_async_copy(d_h.at[pl.ds(0, CH)], dstc, si).wait()

    def issue_gather(kc, rslot, islot):
        srcc, dstc, _ = islot
        ar, br, cr, sg = rslot
        pltpu.async_copy(a_h.at[srcc], ar, sg)
        pltpu.async_copy(b_h.at[dstc], br, sg)
        pltpu.async_copy(c_h.at[pl.ds(tbase + kc * CH, CH)], cr, sg)

    def wait_gather(rslot, islot):
        srcc, dstc, _ = islot
        ar, br, cr, sg = rslot
        pltpu.make_async_copy(a_h.at[srcc], ar, sg).wait()
        pltpu.make_async_copy(b_h.at[dstc], br, sg).wait()
        pltpu.make_async_copy(c_h.at[pl.ds(0, CH)], cr, sg).wait()

    def compute_rows(rslot):
        ar, br, cr, _ = rslot

        def row4(i, rc):
            for rr in range(4):
                r = i * 4 + rr
                for j in range(8):
                    s = pl.ds(j * 16, 16)
                    cr[r, s] = jnp.maximum(ar[r, s] + br[r, s] + cr[r, s], 0.0)
            return rc
        lax.fori_loop(0, CH // 4, row4, 0)

    for j in range(4):
        issue_idx(j, idx_slots[j])
    wait_idx(idx_slots[0])
    issue_gather(0, row_slots[0], idx_slots[0])
    wait_idx(idx_slots[1])
    issue_gather(1, row_slots[1], idx_slots[1])

    def tstep(k6, carry):
        k0 = k6 * NI
        for b in range(NI):
            rslot = row_slots[b % NR]
            islot = idx_slots[b]
            wait_gather(rslot, islot)
            i2 = idx_slots[(b + 2) % NI]
            wait_idx(i2)
            issue_gather(k0 + b + 2, row_slots[(b + 2) % NR], i2)
            compute_rows(rslot)
            pltpu.sync_copy(rslot[2], acc.at[islot[0]], add=True)
            issue_idx(k0 + b + 4, idx_slots[(b + 4) % NI])
        return carry
    lax.fori_loop(0, CPT // NI, tstep, 0)

    for k in range(CPT - CPT % NI, CPT):
        b = k % NI
        rslot = row_slots[b % NR]
        islot = idx_slots[b]
        wait_gather(rslot, islot)
        if k + 2 < CPT:
            i2 = idx_slots[(b + 2) % NI]
            wait_idx(i2)
            issue_gather(k + 2, row_slots[(b + 2) % NR], i2)
        compute_rows(rslot)
        pltpu.sync_copy(rslot[2], acc.at[islot[0]], add=True)

    plsc.subcore_barrier()
    for jj in range(2):
        k = sid + 16 * jj
        @pl.when(k < NZC)
        def _writeout():
            sl = pl.ds(k * ZC, ZC)
            pltpu.sync_copy(acc.at[sl], out_h.at[cid].at[sl])


def _edge_call(a, b, c, src2, dst2):
    mesh = plsc.VectorSubcoreMesh(core_axis_name="c", subcore_axis_name="s")
    scratch = []
    for _ in range(NI):
        scratch += [pltpu.VMEM((CH,), jnp.int32),
                    pltpu.VMEM((CH,), jnp.int32),
                    pltpu.SemaphoreType.DMA]
    for _ in range(NR):
        scratch += [pltpu.VMEM((CH, H), jnp.float32),
                    pltpu.VMEM((CH, H), jnp.float32),
                    pltpu.VMEM((CH, H), jnp.float32),
                    pltpu.SemaphoreType.DMA]
    scratch.append(pltpu.VMEM_SHARED((N, H), jnp.float32))
    f = pl.kernel(
        _edge_body,
        out_type=jax.ShapeDtypeStruct((2, N, H), jnp.float32),
        mesh=mesh,
        scratch_types=scratch,
    )
    return f(a, b, c, src2, dst2)


def _node_call(child, exists, wc, bc2, ws, wd):
    return pl.pallas_call(
        _node_body,
        out_shape=[
            jax.ShapeDtypeStruct((N, H), jnp.float32),
            jax.ShapeDtypeStruct((N, H), jnp.float32),
            jax.ShapeDtypeStruct((1, H), jnp.float32),
            jax.ShapeDtypeStruct((1, 1), jnp.float32),
        ],
    )(child, exists, wc, bc2, ws, wd)


def _c_call(oh, ef, w1, w2, bne2):
    grid = (E // BE,)
    return pl.pallas_call(
        _c_body,
        grid=grid,
        in_specs=[
            pl.BlockSpec((BE, 4), lambda i: (i, 0)),
            pl.BlockSpec((BE, 16), lambda i: (i, 0)),
            pl.BlockSpec((4, H), lambda i: (0, 0)),
            pl.BlockSpec((16, H), lambda i: (0, 0)),
            pl.BlockSpec((1, H), lambda i: (0, 0)),
        ],
        out_specs=pl.BlockSpec((BE, H), lambda i: (i, 0)),
        out_shape=jax.ShapeDtypeStruct((E, H), jnp.float32),
    )(oh, ef, w1, w2, bne2)


def _mid_call(outp, ws, wd):
    return pl.pallas_call(
        _mid_body,
        out_shape=[
            jax.ShapeDtypeStruct((N, H), jnp.float32),
            jax.ShapeDtypeStruct((N, H), jnp.float32),
            jax.ShapeDtypeStruct((1, H), jnp.float32),
        ],
    )(outp, ws, wd)


def _fin_call(outp, s0, s1, es, wp, bp2):
    return pl.pallas_call(
        _fin_body,
        out_shape=jax.ShapeDtypeStruct((1, H), jnp.float32),
    )(outp, s0, s1, es, wp, bp2)


def kernel(child_feats, child_exists, edge_type_onehot, edge_feats, edge_indices,
           Wc, bc, Wne, bne, Wp, bp):
    child = child_feats[0]
    exists = child_exists[0]
    oh = edge_type_onehot[0]
    ef = edge_feats[0]
    src2 = edge_indices[0, :, 0].astype(jnp.int32)
    dst2 = edge_indices[0, :, 1].astype(jnp.int32)
    ws = Wne[:H]
    wd = Wne[H:2 * H]
    w1 = Wne[2 * H:2 * H + 4]
    w2 = Wne[2 * H + 4:]
    bc2 = bc.reshape(1, H)
    bne2 = bne.reshape(1, H)
    bp2 = bp.reshape(1, H)

    a1, b1, s0, es = _node_call(child, exists, Wc, bc2, ws, wd)
    c = _c_call(oh, ef, w1, w2, bne2)
    outp1 = _edge_call(a1, b1, c, src2, dst2)
    a2, b2, s1 = _mid_call(outp1, ws, wd)
    outp2 = _edge_call(a2, b2, c, src2, dst2)
    return _fin_call(outp2, s0, s1, es, Wp, bp2)


# speedup vs baseline: 7.1179x; 1.2008x over previous
import jax
import jax.numpy as jnp
from jax import lax
from jax.experimental import pallas as pl
from jax.experimental.pallas import tpu as pltpu
from jax.experimental.pallas import tpu_sc as plsc

N = 10000
H = 128
E = 320000
CH = 40
NTILES = 32
CPT = E // (NTILES * CH)
NR = 3
NI = 6
ZC = 400
NZC = N // ZC
BE = 3200


def _tdot(lhs_t, rhs):
    return lax.dot_general(lhs_t, rhs, (((0,), (0,)), ((), ())),
                           preferred_element_type=jnp.float32)


def _node_body(child_ref, exists_ref, wc_ref, bc_ref, ws_ref, wd_ref,
               a_ref, b_ref, s0_ref, es_ref):
    cf = _tdot(child_ref[...], wc_ref[...])
    cf = jnp.maximum(cf + bc_ref[...], 0.0) * exists_ref[...]
    a_ref[...] = jnp.dot(cf, ws_ref[...], preferred_element_type=jnp.float32)
    b_ref[...] = jnp.dot(cf, wd_ref[...], preferred_element_type=jnp.float32)
    s0_ref[...] = jnp.sum(cf, axis=0, keepdims=True)
    es_ref[...] = jnp.sum(exists_ref[...], axis=0, keepdims=True)


def _c_body(oh_ref, ef_ref, w1_ref, w2_ref, bne_ref, c_ref):
    c_ref[...] = (_tdot(oh_ref[...], w1_ref[...])
                  + _tdot(ef_ref[...], w2_ref[...])
                  + bne_ref[...])


def _mid_body(outp_ref, ws_ref, wd_ref, a_ref, b_ref, s_ref):
    cf = outp_ref[0] + outp_ref[1]
    a_ref[...] = jnp.dot(cf, ws_ref[...], preferred_element_type=jnp.float32)
    b_ref[...] = jnp.dot(cf, wd_ref[...], preferred_element_type=jnp.float32)
    s_ref[...] = jnp.sum(cf, axis=0, keepdims=True)


def _fin_body(outp_ref, s0_ref, s1_ref, es_ref, wp_ref, bp_ref, o_ref):
    s2 = jnp.sum(outp_ref[0] + outp_ref[1], axis=0, keepdims=True)
    p = jnp.concatenate([s0_ref[...], s1_ref[...], s2], axis=1) / es_ref[0, 0]
    o_ref[...] = jnp.maximum(
        jnp.dot(p, wp_ref[...], preferred_element_type=jnp.float32) + bp_ref[...], 0.0)


def _edge_body(a_h, b_h, c_h, s_h, d_h, out_h, *refs):
    idx_slots = []
    for i in range(NI):
        idx_slots.append(tuple(refs[3 * i:3 * i + 3]))
    row_slots = []
    for i in range(NR):
        row_slots.append(tuple(refs[3 * NI + 4 * i:3 * NI + 4 * i + 4]))
    acc = refs[3 * NI + 4 * NR]

    cid = lax.axis_index("c")
    sid = lax.axis_index("s")
    w = sid * 2 + cid
    tbase = w * CPT * CH
    zero16 = jnp.zeros((16,), jnp.float32)
    zbuf = row_slots[0][0]

    def zrow(r, carry):
        for j in range(8):
            zbuf[r, pl.ds(j * 16, 16)] = zero16
        return carry
    lax.fori_loop(0, CH, zrow, 0)

    for jj in range(2):
        k = sid + 16 * jj
        @pl.when(k < NZC)
        def _zero_chunk():
            for j in range(ZC // CH):
                pltpu.sync_copy(zbuf, acc.at[pl.ds(k * ZC + j * CH, CH)])
    plsc.subcore_barrier()

    def issue_idx(kc, islot):
        srcc, dstc, si = islot
        base = tbase + kc * CH
        pltpu.async_copy(s_h.at[pl.ds(base, CH)], srcc, si)
        pltpu.async_copy(d_h.at[pl.ds(base, CH)], dstc, si)

    def wait_idx(islot):
        srcc, dstc, si = islot
        pltpu.make_async_copy(s_h.at[pl.ds(0, CH)], srcc, si).wait()
        pltpu.make_async_copy(d_h.at[pl.ds(0, CH)], dstc, si).wait()

    def issue_gather(kc, rslot, islot):
        srcc, dstc, _ = islot
        ar, br, cr, sg = rslot
        pltpu.async_copy(a_h.at[srcc], ar, sg)
        pltpu.async_copy(b_h.at[dstc], br, sg)
        pltpu.async_copy(c_h.at[pl.ds(tbase + kc * CH, CH)], cr, sg)

    def wait_gather(rslot, islot):
        srcc, dstc, _ = islot
        ar, br, cr, sg = rslot
        pltpu.make_async_copy(a_h.at[srcc], ar, sg).wait()
        pltpu.make_async_copy(b_h.at[dstc], br, sg).wait()
        pltpu.make_async_copy(c_h.at[pl.ds(0, CH)], cr, sg).wait()

    def compute_rows(rslot):
        ar, br, cr, _ = rslot

        def row4(i, rc):
            for rr in range(4):
                r = i * 4 + rr
                for j in range(8):
                    s = pl.ds(j * 16, 16)
                    cr[r, s] = jnp.maximum(ar[r, s] + br[r, s] + cr[r, s], 0.0)
            return rc
        lax.fori_loop(0, CH // 4, row4, 0)

    for j in range(4):
        issue_idx(j, idx_slots[j])
    wait_idx(idx_slots[0])
    issue_gather(0, row_slots[0], idx_slots[0])
    wait_idx(idx_slots[1])
    issue_gather(1, row_slots[1], idx_slots[1])

    def tstep(k6, carry):
        k0 = k6 * NI
        for b in range(NI):
            rslot = row_slots[b % NR]
            islot = idx_slots[b]
            wait_gather(rslot, islot)
            i2 = idx_slots[(b + 2) % NI]
            wait_idx(i2)
            issue_gather(k0 + b + 2, row_slots[(b + 2) % NR], i2)
            compute_rows(rslot)
            pltpu.sync_copy(rslot[2], acc.at[islot[0]], add=True)
            issue_idx(k0 + b + 4, idx_slots[(b + 4) % NI])
        return carry
    lax.fori_loop(0, CPT // NI, tstep, 0)

    for k in range(CPT - CPT % NI, CPT):
        b = k % NI
        rslot = row_slots[b % NR]
        islot = idx_slots[b]
        wait_gather(rslot, islot)
        if k + 2 < CPT:
            i2 = idx_slots[(b + 2) % NI]
            wait_idx(i2)
            issue_gather(k + 2, row_slots[(b + 2) % NR], i2)
        compute_rows(rslot)
        pltpu.sync_copy(rslot[2], acc.at[islot[0]], add=True)

    plsc.subcore_barrier()
    for jj in range(2):
        k = sid + 16 * jj
        @pl.when(k < NZC)
        def _writeout():
            sl = pl.ds(k * ZC, ZC)
            pltpu.sync_copy(acc.at[sl], out_h.at[cid].at[sl])


def _edge_call(a, b, c, src2, dst2):
    mesh = plsc.VectorSubcoreMesh(core_axis_name="c", subcore_axis_name="s")
    scratch = []
    for _ in range(NI):
        scratch += [pltpu.VMEM((CH,), jnp.int32),
                    pltpu.VMEM((CH,), jnp.int32),
                    pltpu.SemaphoreType.DMA]
    for _ in range(NR):
        scratch += [pltpu.VMEM((CH, H), jnp.float32),
                    pltpu.VMEM((CH, H), jnp.float32),
                    pltpu.VMEM((CH, H), jnp.float32),
                    pltpu.SemaphoreType.DMA]
    scratch.append(pltpu.VMEM_SHARED((N, H), jnp.float32))
    f = pl.kernel(
        _edge_body,
        out_type=jax.ShapeDtypeStruct((2, N, H), jnp.float32),
        mesh=mesh,
        scratch_types=scratch,
    )
    return f(a, b, c, src2, dst2)


def _node_call(child_t, exists, wc, bc2, ws, wd):
    return pl.pallas_call(
        _node_body,
        out_shape=[
            jax.ShapeDtypeStruct((N, H), jnp.float32),
            jax.ShapeDtypeStruct((N, H), jnp.float32),
            jax.ShapeDtypeStruct((1, H), jnp.float32),
            jax.ShapeDtypeStruct((1, 1), jnp.float32),
        ],
    )(child_t, exists, wc, bc2, ws, wd)


def _c_call(oh_t, ef_t, w1, w2, bne2):
    grid = (E // BE,)
    return pl.pallas_call(
        _c_body,
        grid=grid,
        in_specs=[
            pl.BlockSpec((4, BE), lambda i: (0, i)),
            pl.BlockSpec((16, BE), lambda i: (0, i)),
            pl.BlockSpec((4, H), lambda i: (0, 0)),
            pl.BlockSpec((16, H), lambda i: (0, 0)),
            pl.BlockSpec((1, H), lambda i: (0, 0)),
        ],
        out_specs=pl.BlockSpec((BE, H), lambda i: (i, 0)),
        out_shape=jax.ShapeDtypeStruct((E, H), jnp.float32),
    )(oh_t, ef_t, w1, w2, bne2)


def _mid_call(outp, ws, wd):
    return pl.pallas_call(
        _mid_body,
        out_shape=[
            jax.ShapeDtypeStruct((N, H), jnp.float32),
            jax.ShapeDtypeStruct((N, H), jnp.float32),
            jax.ShapeDtypeStruct((1, H), jnp.float32),
        ],
    )(outp, ws, wd)


def _fin_call(outp, s0, s1, es, wp, bp2):
    return pl.pallas_call(
        _fin_body,
        out_shape=jax.ShapeDtypeStruct((1, H), jnp.float32),
    )(outp, s0, s1, es, wp, bp2)


def kernel(child_feats, child_exists, edge_type_onehot, edge_feats, edge_indices,
           Wc, bc, Wne, bne, Wp, bp):
    child_t = jnp.transpose(child_feats, (0, 2, 1))[0]
    exists = child_exists[0]
    oh_t = jnp.transpose(edge_type_onehot, (0, 2, 1))[0]
    ef_t = jnp.transpose(edge_feats, (0, 2, 1))[0]
    src2 = edge_indices[0, :, 0].astype(jnp.int32)
    dst2 = edge_indices[0, :, 1].astype(jnp.int32)
    ws = Wne[:H]
    wd = Wne[H:2 * H]
    w1 = Wne[2 * H:2 * H + 4]
    w2 = Wne[2 * H + 4:]
    bc2 = bc.reshape(1, H)
    bne2 = bne.reshape(1, H)
    bp2 = bp.reshape(1, H)

    a1, b1, s0, es = _node_call(child_t, exists, Wc, bc2, ws, wd)
    c = _c_call(oh_t, ef_t, w1, w2, bne2)
    outp1 = _edge_call(a1, b1, c, src2, dst2)
    a2, b2, s1 = _mid_call(outp1, ws, wd)
    outp2 = _edge_call(a2, b2, c, src2, dst2)
    return _fin_call(outp2, s0, s1, es, Wp, bp2)


# speedup vs baseline: 7.1481x; 1.0042x over previous
import jax
import jax.numpy as jnp
from jax import lax
from jax.experimental import pallas as pl
from jax.experimental.pallas import tpu as pltpu
from jax.experimental.pallas import tpu_sc as plsc

N = 10000
H = 128
E = 320000
CH = 40
NTILES = 32
CPT = E // (NTILES * CH)
NR = 3
NI = 6
ZC = 400
NZC = N // ZC
BE = 3200


def _tdot(lhs_t, rhs):
    return lax.dot_general(lhs_t, rhs, (((0,), (0,)), ((), ())),
                           preferred_element_type=jnp.float32)


def _node_body(child_ref, exists_ref, wc_ref, bc_ref, ws_ref, wd_ref,
               a_ref, b_ref, s0_ref, es_ref):
    cf = _tdot(child_ref[...], wc_ref[...])
    cf = jnp.maximum(cf + bc_ref[...], 0.0) * exists_ref[...]
    a_ref[...] = jnp.dot(cf, ws_ref[...], preferred_element_type=jnp.float32)
    b_ref[...] = jnp.dot(cf, wd_ref[...], preferred_element_type=jnp.float32)
    s0_ref[...] = jnp.sum(cf, axis=0, keepdims=True)
    es_ref[...] = jnp.sum(exists_ref[...], axis=0, keepdims=True)


def _c_body(oh_ref, ef_ref, w1_ref, w2_ref, bne_ref, c_ref):
    c_ref[...] = (_tdot(oh_ref[...], w1_ref[...])
                  + _tdot(ef_ref[...], w2_ref[...])
                  + bne_ref[...])


def _mid_body(outp_ref, ws_ref, wd_ref, a_ref, b_ref, s_ref):
    cf = outp_ref[0] + outp_ref[1]
    a_ref[...] = jnp.dot(cf, ws_ref[...], preferred_element_type=jnp.float32)
    b_ref[...] = jnp.dot(cf, wd_ref[...], preferred_element_type=jnp.float32)
    s_ref[...] = jnp.sum(cf, axis=0, keepdims=True)


def _fin_body(outp_ref, s0_ref, s1_ref, es_ref, wp_ref, bp_ref, o_ref):
    s2 = jnp.sum(outp_ref[0] + outp_ref[1], axis=0, keepdims=True)
    p = jnp.concatenate([s0_ref[...], s1_ref[...], s2], axis=1) / es_ref[0, 0]
    o_ref[...] = jnp.maximum(
        jnp.dot(p, wp_ref[...], preferred_element_type=jnp.float32) + bp_ref[...], 0.0)


def _edge_body(a_h, b_h, c_h, s_h, d_h, out_h, *refs):
    idx_slots = []
    for i in range(NI):
        idx_slots.append(tuple(refs[3 * i:3 * i + 3]))
    row_slots = []
    for i in range(NR):
        row_slots.append(tuple(refs[3 * NI + 5 * i:3 * NI + 5 * i + 5]))
    acc = refs[3 * NI + 5 * NR]

    cid = lax.axis_index("c")
    sid = lax.axis_index("s")
    w = sid * 2 + cid
    tbase = w * CPT * CH
    zero16 = jnp.zeros((16,), jnp.float32)
    zbuf = row_slots[0][0]

    def zrow(r, carry):
        for j in range(8):
            zbuf[r, pl.ds(j * 16, 16)] = zero16
        return carry
    lax.fori_loop(0, CH, zrow, 0)

    for jj in range(2):
        k = sid + 16 * jj
        @pl.when(k < NZC)
        def _zero_chunk():
            for j in range(ZC // CH):
                pltpu.sync_copy(zbuf, acc.at[pl.ds(k * ZC + j * CH, CH)])
    plsc.subcore_barrier()

    def issue_idx(kc, islot):
        srcc, dstc, si = islot
        base = tbase + kc * CH
        pltpu.async_copy(s_h.at[pl.ds(base, CH)], srcc, si)
        pltpu.async_copy(d_h.at[pl.ds(base, CH)], dstc, si)

    def wait_idx(islot):
        srcc, dstc, si = islot
        pltpu.make_async_copy(s_h.at[pl.ds(0, CH)], srcc, si).wait()
        pltpu.make_async_copy(d_h.at[pl.ds(0, CH)], dstc, si).wait()

    def issue_gather(kc, rslot, islot):
        srcc, dstc, _ = islot
        ar, br, cr, sg, _ = rslot
        pltpu.async_copy(a_h.at[srcc], ar, sg)
        pltpu.async_copy(b_h.at[dstc], br, sg)
        pltpu.async_copy(c_h.at[pl.ds(tbase + kc * CH, CH)], cr, sg)

    def wait_gather(rslot, islot):
        srcc, dstc, _ = islot
        ar, br, cr, sg, _ = rslot
        pltpu.make_async_copy(a_h.at[srcc], ar, sg).wait()
        pltpu.make_async_copy(b_h.at[dstc], br, sg).wait()
        pltpu.make_async_copy(c_h.at[pl.ds(0, CH)], cr, sg).wait()

    def issue_scatter(rslot, islot):
        pltpu.async_copy(rslot[2], acc.at[islot[0]], rslot[4], add=True)

    def wait_scatter(rslot):
        pltpu.make_async_copy(rslot[2], acc.at[idx_slots[0][0]], rslot[4]).wait()

    def compute_rows(rslot):
        ar, br, cr, _, _ = rslot

        def row4(i, rc):
            for rr in range(4):
                r = i * 4 + rr
                for j in range(8):
                    s = pl.ds(j * 16, 16)
                    cr[r, s] = jnp.maximum(ar[r, s] + br[r, s] + cr[r, s], 0.0)
            return rc
        lax.fori_loop(0, CH // 4, row4, 0)

    def chunk_step(k, b6, gather_next=True, idx_next=True, scatter_wait=True):
        rslot = row_slots[b6 % NR]
        islot = idx_slots[b6]
        wait_gather(rslot, islot)
        if gather_next:
            nslot = row_slots[(b6 + 2) % NR]
            if scatter_wait:
                wait_scatter(nslot)
            i2 = idx_slots[(b6 + 2) % NI]
            wait_idx(i2)
            issue_gather(k + 2, nslot, i2)
        compute_rows(rslot)
        issue_scatter(rslot, islot)
        if idx_next:
            issue_idx(k + 4, idx_slots[(b6 + 4) % NI])

    for j in range(4):
        issue_idx(j, idx_slots[j])
    wait_idx(idx_slots[0])
    issue_gather(0, row_slots[0], idx_slots[0])
    wait_idx(idx_slots[1])
    issue_gather(1, row_slots[1], idx_slots[1])

    chunk_step(0, 0, scatter_wait=False)
    chunk_step(1, 1)

    def tstep(k6, carry):
        k0 = 2 + k6 * NI
        for b in range(NI):
            chunk_step(k0 + b, (2 + b) % NI)
        return carry
    lax.fori_loop(0, (CPT - 2 - 8) // NI, tstep, 0)

    for k in range(CPT - 8, CPT):
        chunk_step(k, k % NI, gather_next=(k + 2 < CPT), idx_next=(k + 4 < CPT))

    for k in range(CPT - NR, CPT):
        wait_scatter(row_slots[k % NR])

    plsc.subcore_barrier()
    for jj in range(2):
        k = sid + 16 * jj
        @pl.when(k < NZC)
        def _writeout():
            sl = pl.ds(k * ZC, ZC)
            pltpu.sync_copy(acc.at[sl], out_h.at[cid].at[sl])


def _edge_call(a, b, c, src2, dst2):
    mesh = plsc.VectorSubcoreMesh(core_axis_name="c", subcore_axis_name="s")
    scratch = []
    for _ in range(NI):
        scratch += [pltpu.VMEM((CH,), jnp.int32),
                    pltpu.VMEM((CH,), jnp.int32),
                    pltpu.SemaphoreType.DMA]
    for _ in range(NR):
        scratch += [pltpu.VMEM((CH, H), jnp.float32),
                    pltpu.VMEM((CH, H), jnp.float32),
                    pltpu.VMEM((CH, H), jnp.float32),
                    pltpu.SemaphoreType.DMA,
                    pltpu.SemaphoreType.DMA]
    scratch.append(pltpu.VMEM_SHARED((N, H), jnp.float32))
    f = pl.kernel(
        _edge_body,
        out_type=jax.ShapeDtypeStruct((2, N, H), jnp.float32),
        mesh=mesh,
        scratch_types=scratch,
    )
    return f(a, b, c, src2, dst2)


def _node_call(child_t, exists, wc, bc2, ws, wd):
    return pl.pallas_call(
        _node_body,
        out_shape=[
            jax.ShapeDtypeStruct((N, H), jnp.float32),
            jax.ShapeDtypeStruct((N, H), jnp.float32),
            jax.ShapeDtypeStruct((1, H), jnp.float32),
            jax.ShapeDtypeStruct((1, 1), jnp.float32),
        ],
    )(child_t, exists, wc, bc2, ws, wd)


def _c_call(oh_t, ef_t, w1, w2, bne2):
    grid = (E // BE,)
    return pl.pallas_call(
        _c_body,
        grid=grid,
        in_specs=[
            pl.BlockSpec((4, BE), lambda i: (0, i)),
            pl.BlockSpec((16, BE), lambda i: (0, i)),
            pl.BlockSpec((4, H), lambda i: (0, 0)),
            pl.BlockSpec((16, H), lambda i: (0, 0)),
            pl.BlockSpec((1, H), lambda i: (0, 0)),
        ],
        out_specs=pl.BlockSpec((BE, H), lambda i: (i, 0)),
        out_shape=jax.ShapeDtypeStruct((E, H), jnp.float32),
    )(oh_t, ef_t, w1, w2, bne2)


def _mid_call(outp, ws, wd):
    return pl.pallas_call(
        _mid_body,
        out_shape=[
            jax.ShapeDtypeStruct((N, H), jnp.float32),
            jax.ShapeDtypeStruct((N, H), jnp.float32),
            jax.ShapeDtypeStruct((1, H), jnp.float32),
        ],
    )(outp, ws, wd)


def _fin_call(outp, s0, s1, es, wp, bp2):
    return pl.pallas_call(
        _fin_body,
        out_shape=jax.ShapeDtypeStruct((1, H), jnp.float32),
    )(outp, s0, s1, es, wp, bp2)


def kernel(child_feats, child_exists, edge_type_onehot, edge_feats, edge_indices,
           Wc, bc, Wne, bne, Wp, bp):
    child_t = jnp.transpose(child_feats, (0, 2, 1))[0]
    exists = child_exists[0]
    oh_t = jnp.transpose(edge_type_onehot, (0, 2, 1))[0]
    ef_t = jnp.transpose(edge_feats, (0, 2, 1))[0]
    src2 = edge_indices[0, :, 0].astype(jnp.int32)
    dst2 = edge_indices[0, :, 1].astype(jnp.int32)
    ws = Wne[:H]
    wd = Wne[H:2 * H]
    w1 = Wne[2 * H:2 * H + 4]
    w2 = Wne[2 * H + 4:]
    bc2 = bc.reshape(1, H)
    bne2 = bne.reshape(1, H)
    bp2 = bp.reshape(1, H)

    a1, b1, s0, es = _node_call(child_t, exists, Wc, bc2, ws, wd)
    c = _c_call(oh_t, ef_t, w1, w2, bne2)
    outp1 = _edge_call(a1, b1, c, src2, dst2)
    a2, b2, s1 = _mid_call(outp1, ws, wd)
    outp2 = _edge_call(a2, b2, c, src2, dst2)
    return _fin_call(outp2, s0, s1, es, Wp, bp2)
